# Initial kernel scaffold; baseline (speedup 1.0000x reference)
import jax
import jax.numpy as jnp
from jax.experimental import pallas as pl


def kernel(x, gate_w, gate_b, temperature, ek_w, ek_b, ev_w, ev_b, sk_w, sk_b, sv_w, sv_b):
    raise NotImplementedError("write your pallas kernel here")


# speedup vs baseline: 1.6745x; 1.6745x over previous
import functools

import jax
import jax.numpy as jnp
from jax.experimental import pallas as pl
from jax.experimental.pallas import tpu as pltpu

D_MODEL = 1024
HIDDEN = 2048
NUM_EXPERTS = 8


def _router_body(x_ref, gw_ref, gb_ref, temp_ref, probs_ref):
    x_bf = x_ref[...].astype(jnp.bfloat16)
    gw_bf = gw_ref[...].astype(jnp.bfloat16)
    logits = jnp.dot(x_bf, gw_bf, preferred_element_type=jnp.float32)
    logits = logits + gb_ref[...]
    safe_temp = jnp.maximum(temp_ref[0, 0], 0.1)
    logits = logits / safe_temp
    m = jnp.max(logits, axis=-1, keepdims=True)
    e = jnp.exp(logits - m)
    probs_ref[...] = e / jnp.sum(e, axis=-1, keepdims=True)


def _router(x2, gate_w, gate_b, temperature):
    T = x2.shape[0]
    return pl.pallas_call(
        _router_body,
        out_shape=jax.ShapeDtypeStruct((T, NUM_EXPERTS), jnp.float32),
    )(x2, gate_w, gate_b.reshape(1, NUM_EXPERTS), temperature.reshape(1, 1))


def _expert_body(xs_ref, ekw_ref, ekb_ref, evw_ref, evb_ref, sc_ref, out_ref):
    xs_bf = xs_ref[...].astype(jnp.bfloat16)
    kw_bf = ekw_ref[0].astype(jnp.bfloat16)
    h = jnp.dot(xs_bf, kw_bf, preferred_element_type=jnp.float32)
    h = jax.nn.gelu(h + ekb_ref[0])
    vw_bf = evw_ref[0].astype(jnp.bfloat16)
    o = jnp.dot(h.astype(jnp.bfloat16), vw_bf, preferred_element_type=jnp.float32)
    o = o + evb_ref[0]
    out_ref[...] = o * sc_ref[0, 0][:, None]


def _routed_ffn(xs, ek_w, ek_b, ev_w, ev_b, scores, cap):
    E = NUM_EXPERTS
    return pl.pallas_call(
        _expert_body,
        grid=(E,),
        in_specs=[
            pl.BlockSpec((cap, D_MODEL), lambda e: (e, 0)),
            pl.BlockSpec((1, D_MODEL, HIDDEN), lambda e: (e, 0, 0)),
            pl.BlockSpec((1, 1, HIDDEN), lambda e: (e, 0, 0)),
            pl.BlockSpec((1, HIDDEN, D_MODEL), lambda e: (e, 0, 0)),
            pl.BlockSpec((1, 1, D_MODEL), lambda e: (e, 0, 0)),
            pl.BlockSpec((1, 1, cap), lambda e: (e, 0, 0)),
        ],
        out_specs=pl.BlockSpec((cap, D_MODEL), lambda e: (e, 0)),
        out_shape=jax.ShapeDtypeStruct((E * cap, D_MODEL), jnp.float32),
    )(xs, ek_w, ek_b.reshape(E, 1, HIDDEN), ev_w,
      ev_b.reshape(E, 1, D_MODEL), scores.reshape(E, 1, cap))


def _shared_body(x_ref, skw_ref, skb_ref, svw_ref, svb_ref, out_ref):
    x_bf = x_ref[...].astype(jnp.bfloat16)
    kw_bf = skw_ref[...].astype(jnp.bfloat16)
    h = jnp.dot(x_bf, kw_bf, preferred_element_type=jnp.float32)
    h = jax.nn.gelu(h + skb_ref[...])
    vw_bf = svw_ref[...].astype(jnp.bfloat16)
    o = jnp.dot(h.astype(jnp.bfloat16), vw_bf, preferred_element_type=jnp.float32)
    out_ref[...] = o + svb_ref[...]


def _shared_ffn(x2, sk_w, sk_b, sv_w, sv_b, blk):
    T = x2.shape[0]
    return pl.pallas_call(
        _shared_body,
        grid=(T // blk,),
        in_specs=[
            pl.BlockSpec((blk, D_MODEL), lambda i: (i, 0)),
            pl.BlockSpec((D_MODEL, HIDDEN), lambda i: (0, 0)),
            pl.BlockSpec((1, HIDDEN), lambda i: (0, 0)),
            pl.BlockSpec((HIDDEN, D_MODEL), lambda i: (0, 0)),
            pl.BlockSpec((1, D_MODEL), lambda i: (0, 0)),
        ],
        out_specs=pl.BlockSpec((blk, D_MODEL), lambda i: (i, 0)),
        out_shape=jax.ShapeDtypeStruct((T, D_MODEL), jnp.float32),
    )(x2, sk_w, sk_b.reshape(1, HIDDEN), sv_w, sv_b.reshape(1, D_MODEL))


def kernel(x, gate_w, gate_b, temperature, ek_w, ek_b, ev_w, ev_b,
           sk_w, sk_b, sv_w, sv_b):
    B, S, D = x.shape
    T = B * S
    cap = max(8, T // NUM_EXPERTS)
    x2 = x.reshape(T, D)

    probs = _router(x2, gate_w, gate_b, temperature)
    flat_probs = probs.T
    scores, token_idx = jax.lax.top_k(flat_probs, k=cap)

    xs = jnp.take(x2, token_idx.reshape(-1), axis=0)
    slots = _routed_ffn(xs, ek_w, ek_b, ev_w, ev_b, scores, cap)

    shared = _shared_ffn(x2, sk_w, sk_b, sv_w, sv_b, blk=512)

    out = shared.at[token_idx.reshape(-1)].add(slots)
    return out.reshape(B, S, D)


# speedup vs baseline: 1.7883x; 1.0680x over previous
import dataclasses
import functools

import jax
import jax.numpy as jnp
from jax import lax
from jax.experimental import pallas as pl
from jax.experimental.pallas import tpu as pltpu
from jax.experimental.pallas import tpu_sc as plsc

D_MODEL = 1024
HIDDEN = 2048
NUM_EXPERTS = 8
ONE_BITS = 0x3F800001


def _router_body(cap, x_ref, gw_ref, gb_ref, temp_ref, probs_ref, thr_ref):
    x_bf = x_ref[...].astype(jnp.bfloat16)
    gw_bf = gw_ref[...].astype(jnp.bfloat16)
    logits = lax.dot_general(gw_bf, x_bf, (((0,), (1,)), ((), ())),
                             preferred_element_type=jnp.float32)
    logits = logits + gb_ref[...]
    safe_temp = jnp.maximum(temp_ref[0, 0], 0.1)
    logits = logits / safe_temp
    m = jnp.max(logits, axis=0, keepdims=True)
    ex = jnp.exp(logits - m)
    probs = ex / jnp.sum(ex, axis=0, keepdims=True)
    probs_ref[...] = probs

    bits = lax.bitcast_convert_type(probs, jnp.int32)

    def it(_, lohi):
        lo, hi = lohi
        mid = lo + (hi - lo) // 2
        cnt = jnp.sum((bits >= mid).astype(jnp.int32), axis=1, keepdims=True)
        ge = cnt >= cap
        return jnp.where(ge, mid, lo), jnp.where(ge, hi, mid)

    lo0 = jnp.zeros((NUM_EXPERTS, 1), jnp.int32)
    hi0 = jnp.full((NUM_EXPERTS, 1), ONE_BITS, jnp.int32)
    lo, _ = lax.fori_loop(0, 31, it, (lo0, hi0))
    thr_ref[...] = jnp.broadcast_to(lo, (NUM_EXPERTS, 128))


def _router(x2, gate_w, gate_b, temperature, cap):
    T = x2.shape[0]
    return pl.pallas_call(
        functools.partial(_router_body, cap),
        out_shape=[jax.ShapeDtypeStruct((NUM_EXPERTS, T), jnp.float32),
                   jax.ShapeDtypeStruct((NUM_EXPERTS, 128), jnp.int32)],
    )(x2, gate_w, gate_b.reshape(NUM_EXPERTS, 1), temperature.reshape(1, 1))


_GCHUNK = 64


def _topk_gather_sc(probs_t, thresh16, x2, cap):
    T, D = x2.shape
    E = NUM_EXPERTS
    epc = E // 2
    mesh = plsc.VectorSubcoreMesh(core_axis_name="c", subcore_axis_name="s")

    cp = pltpu.CompilerParams()
    if "needs_layout_passes" in pltpu.CompilerParams.__dataclass_fields__:
        cp = dataclasses.replace(cp, needs_layout_passes=False)

    @functools.partial(
        pl.kernel,
        compiler_params=cp,
        out_type=[jax.ShapeDtypeStruct((E * cap, D), jnp.float32),
                  jax.ShapeDtypeStruct((E, cap), jnp.int32),
                  jax.ShapeDtypeStruct((E, cap), jnp.float32)],
        mesh=mesh,
        scratch_types=[
            pltpu.VMEM((T,), jnp.float32),
            pltpu.VMEM((cap + 32,), jnp.int32),
            pltpu.VMEM((cap + 32,), jnp.float32),
            pltpu.VMEM((16,), jnp.int32),
            pltpu.VMEM((_GCHUNK,), jnp.int32),
            pltpu.VMEM((_GCHUNK, D), jnp.float32),
            pltpu.SemaphoreType.DMA,
        ],
    )
    def k(probs_hbm, th_hbm, x_hbm, xs_hbm, idx_hbm, sc_hbm,
          pf_v, si_v, sv_v, th_v, ic_v, rows_v, sem):
        c = lax.axis_index("c")
        s = lax.axis_index("s")

        @pl.when(s < epc)
        def _():
            e = c * epc + s
            pltpu.sync_copy(probs_hbm.at[e], pf_v)
            pltpu.sync_copy(th_hbm, th_v)
            evec = jnp.full((16,), e, jnp.int32)
            tvec = plsc.load_gather(th_v, [evec])
            tfvec = plsc.bitcast(tvec, jnp.float32)
            lanes = jnp.arange(16, dtype=jnp.int32)

            def pass1(j, off):
                pf = pf_v[pl.ds(j * 16, 16)]
                pb = plsc.bitcast(pf, jnp.int32)
                msk = pb > tvec
                cnt = jnp.max(plsc.all_reduce_population_count(msk))
                iv = lanes + j * 16
                plsc.store_compressed(si_v.at[pl.ds(off, 16)], iv, mask=msk)
                plsc.store_compressed(sv_v.at[pl.ds(off, 16)], pf, mask=msk)
                return off + cnt

            ngt = lax.fori_loop(0, T // 16, pass1, 0)

            def pass2(j, off):
                pf = pf_v[pl.ds(j * 16, 16)]
                pb = plsc.bitcast(pf, jnp.int32)
                msk = pb == tvec
                cnt = jnp.max(plsc.all_reduce_population_count(msk))
                iv = lanes + j * 16

                @pl.when(off < cap)
                def _():
                    plsc.store_compressed(si_v.at[pl.ds(off, 16)], iv, mask=msk)
                    plsc.store_compressed(sv_v.at[pl.ds(off, 16)], tfvec, mask=msk)

                return off + jnp.where(off < cap, cnt, 0)

            lax.fori_loop(0, T // 16, pass2, ngt)

            pltpu.sync_copy(si_v.at[pl.ds(0, cap)], idx_hbm.at[e])
            pltpu.sync_copy(sv_v.at[pl.ds(0, cap)], sc_hbm.at[e])

            def gath(g, carry):
                for jj in range(_GCHUNK // 16):
                    ic_v[pl.ds(jj * 16, 16)] = (
                        si_v[pl.ds(g * _GCHUNK + jj * 16, 16)])
                pltpu.async_copy(x_hbm.at[ic_v], rows_v, sem).wait()
                pltpu.sync_copy(
                    rows_v, xs_hbm.at[pl.ds(e * cap + g * _GCHUNK, _GCHUNK)])
                return carry

            lax.fori_loop(0, cap // _GCHUNK, gath, 0)

    return k(probs_t, thresh16, x2)


def _expert_body(xs_ref, ekw_ref, ekb_ref, evw_ref, evb_ref, sc_ref, out_ref):
    xs_bf = xs_ref[...].astype(jnp.bfloat16)
    kw_bf = ekw_ref[0].astype(jnp.bfloat16)
    h = jnp.dot(xs_bf, kw_bf, preferred_element_type=jnp.float32)
    h = jax.nn.gelu(h + ekb_ref[0])
    vw_bf = evw_ref[0].astype(jnp.bfloat16)
    o = jnp.dot(h.astype(jnp.bfloat16), vw_bf, preferred_element_type=jnp.float32)
    o = o + evb_ref[0]
    out_ref[...] = o * sc_ref[0, 0][:, None]


def _routed_ffn(xs, ek_w, ek_b, ev_w, ev_b, scores, cap):
    E = NUM_EXPERTS
    return pl.pallas_call(
        _expert_body,
        grid=(E,),
        in_specs=[
            pl.BlockSpec((cap, D_MODEL), lambda e: (e, 0)),
            pl.BlockSpec((1, D_MODEL, HIDDEN), lambda e: (e, 0, 0)),
            pl.BlockSpec((1, 1, HIDDEN), lambda e: (e, 0, 0)),
            pl.BlockSpec((1, HIDDEN, D_MODEL), lambda e: (e, 0, 0)),
            pl.BlockSpec((1, 1, D_MODEL), lambda e: (e, 0, 0)),
            pl.BlockSpec((1, 1, cap), lambda e: (e, 0, 0)),
        ],
        out_specs=pl.BlockSpec((cap, D_MODEL), lambda e: (e, 0)),
        out_shape=jax.ShapeDtypeStruct((E * cap, D_MODEL), jnp.float32),
    )(xs, ek_w, ek_b.reshape(E, 1, HIDDEN), ev_w,
      ev_b.reshape(E, 1, D_MODEL), scores.reshape(E, 1, cap))


def _shared_body(x_ref, skw_ref, skb_ref, svw_ref, svb_ref, out_ref):
    x_bf = x_ref[...].astype(jnp.bfloat16)
    kw_bf = skw_ref[...].astype(jnp.bfloat16)
    h = jnp.dot(x_bf, kw_bf, preferred_element_type=jnp.float32)
    h = jax.nn.gelu(h + skb_ref[...])
    vw_bf = svw_ref[...].astype(jnp.bfloat16)
    o = jnp.dot(h.astype(jnp.bfloat16), vw_bf, preferred_element_type=jnp.float32)
    out_ref[...] = o + svb_ref[...]


def _shared_ffn(x2, sk_w, sk_b, sv_w, sv_b, blk):
    T = x2.shape[0]
    return pl.pallas_call(
        _shared_body,
        grid=(T // blk,),
        in_specs=[
            pl.BlockSpec((blk, D_MODEL), lambda i: (i, 0)),
            pl.BlockSpec((D_MODEL, HIDDEN), lambda i: (0, 0)),
            pl.BlockSpec((1, HIDDEN), lambda i: (0, 0)),
            pl.BlockSpec((HIDDEN, D_MODEL), lambda i: (0, 0)),
            pl.BlockSpec((1, D_MODEL), lambda i: (0, 0)),
        ],
        out_specs=pl.BlockSpec((blk, D_MODEL), lambda i: (i, 0)),
        out_shape=jax.ShapeDtypeStruct((T, D_MODEL), jnp.float32),
    )(x2, sk_w, sk_b.reshape(1, HIDDEN), sv_w, sv_b.reshape(1, D_MODEL))


def kernel(x, gate_w, gate_b, temperature, ek_w, ek_b, ev_w, ev_b,
           sk_w, sk_b, sv_w, sv_b):
    B, S, D = x.shape
    T = B * S
    cap = max(8, T // NUM_EXPERTS)
    x2 = x.reshape(T, D)

    probs_t, thr = _router(x2, gate_w, gate_b, temperature, cap)
    thresh16 = jnp.pad(thr[:, 0], (0, 16 - NUM_EXPERTS))

    xs, token_idx, scores = _topk_gather_sc(probs_t, thresh16, x2, cap)

    slots = _routed_ffn(xs, ek_w, ek_b, ev_w, ev_b, scores, cap)
    shared = _shared_ffn(x2, sk_w, sk_b, sv_w, sv_b, blk=512)

    out = shared.at[token_idx.reshape(-1)].add(slots)
    return out.reshape(B, S, D)


# speedup vs baseline: 1.8752x; 1.0485x over previous
import dataclasses
import functools

import jax
import jax.numpy as jnp
from jax import lax
from jax.experimental import pallas as pl
from jax.experimental.pallas import tpu as pltpu
from jax.experimental.pallas import tpu_sc as plsc

D_MODEL = 1024
HIDDEN = 2048
NUM_EXPERTS = 8
ONE_BITS = 0x3F800001


def _router_body(cap, x_ref, gw_ref, gb_ref, temp_ref, probs_ref, thr_ref):
    x_bf = x_ref[...].astype(jnp.bfloat16)
    gw_bf = gw_ref[...].astype(jnp.bfloat16)
    logits = lax.dot_general(gw_bf, x_bf, (((0,), (1,)), ((), ())),
                             preferred_element_type=jnp.float32)
    logits = logits + gb_ref[...]
    safe_temp = jnp.maximum(temp_ref[0, 0], 0.1)
    logits = logits / safe_temp
    m = jnp.max(logits, axis=0, keepdims=True)
    ex = jnp.exp(logits - m)
    probs = ex / jnp.sum(ex, axis=0, keepdims=True)
    probs_ref[...] = probs

    bits = lax.bitcast_convert_type(probs, jnp.int32)

    def it(_, lohi):
        lo, hi = lohi
        mid = lo + (hi - lo) // 2
        cnt = jnp.sum((bits >= mid).astype(jnp.int32), axis=1, keepdims=True)
        ge = cnt >= cap
        return jnp.where(ge, mid, lo), jnp.where(ge, hi, mid)

    lo0 = jnp.zeros((NUM_EXPERTS, 1), jnp.int32)
    hi0 = jnp.full((NUM_EXPERTS, 1), ONE_BITS, jnp.int32)
    lo, _ = lax.fori_loop(0, 31, it, (lo0, hi0))
    thr_ref[...] = jnp.broadcast_to(lo, (NUM_EXPERTS, 128))


def _router(x2, gate_w, gate_b, temperature, cap):
    T = x2.shape[0]
    return pl.pallas_call(
        functools.partial(_router_body, cap),
        out_shape=[jax.ShapeDtypeStruct((NUM_EXPERTS, T), jnp.float32),
                   jax.ShapeDtypeStruct((NUM_EXPERTS, 128), jnp.int32)],
    )(x2, gate_w, gate_b.reshape(NUM_EXPERTS, 1), temperature.reshape(1, 1))


_GCHUNK = 64


def _topk_gather_sc(probs_t, thresh16, x2, cap):
    T, D = x2.shape
    E = NUM_EXPERTS
    epc = E // 2
    mesh = plsc.VectorSubcoreMesh(core_axis_name="c", subcore_axis_name="s")

    cp = pltpu.CompilerParams()
    if "needs_layout_passes" in pltpu.CompilerParams.__dataclass_fields__:
        cp = dataclasses.replace(cp, needs_layout_passes=False)

    @functools.partial(
        pl.kernel,
        compiler_params=cp,
        out_type=[jax.ShapeDtypeStruct((E * cap, D), jnp.float32),
                  jax.ShapeDtypeStruct((E, cap), jnp.int32),
                  jax.ShapeDtypeStruct((E, cap), jnp.float32)],
        mesh=mesh,
        scratch_types=[
            pltpu.VMEM((T,), jnp.float32),
            pltpu.VMEM((cap + 32,), jnp.int32),
            pltpu.VMEM((cap + 32,), jnp.float32),
            pltpu.VMEM((cap + 32,), jnp.int32),
            pltpu.VMEM((16,), jnp.int32),
            pltpu.VMEM((2 * _GCHUNK,), jnp.int32),
            pltpu.VMEM((_GCHUNK, D), jnp.float32),
            pltpu.SemaphoreType.DMA,
        ],
    )
    def k(probs_hbm, th_hbm, x_hbm, xs_hbm, idx_hbm, sc_hbm,
          pf_v, si_v, sv_v, se_v, th_v, ic_v, rows_v, sem):
        c = lax.axis_index("c")
        s = lax.axis_index("s")

        @pl.when(s < epc)
        def _():
            e = c * epc + s
            pltpu.sync_copy(probs_hbm.at[e], pf_v)
            pltpu.sync_copy(th_hbm, th_v)
            evec = jnp.full((16,), e, jnp.int32)
            tvec = plsc.load_gather(th_v, [evec])
            tfvec = plsc.bitcast(tvec, jnp.float32)
            lanes = jnp.arange(16, dtype=jnp.int32)

            def scan_chunk(j4, carry):
                off, eoff = carry
                for u in range(4):
                    j = j4 * 4 + u
                    pf = pf_v[pl.ds(j * 16, 16)]
                    pb = plsc.bitcast(pf, jnp.int32)
                    iv = lanes + j * 16
                    mg = pb > tvec
                    cg = jnp.max(plsc.all_reduce_population_count(mg))
                    plsc.store_compressed(si_v.at[pl.ds(off, 16)], iv, mask=mg)
                    plsc.store_compressed(sv_v.at[pl.ds(off, 16)], pf, mask=mg)
                    off = off + cg
                    me = pb == tvec
                    ce = jnp.max(plsc.all_reduce_population_count(me))

                    @pl.when(eoff < cap)
                    def _():
                        plsc.store_compressed(se_v.at[pl.ds(eoff, 16)], iv,
                                              mask=me)

                    eoff = eoff + jnp.where(eoff < cap, ce, 0)
                return off, eoff

            ngt, _ = lax.fori_loop(0, T // 64, scan_chunk, (0, 0))

            def fill(carry):
                j, off = carry
                si_v[pl.ds(off, 16)] = se_v[pl.ds(j * 16, 16)]
                sv_v[pl.ds(off, 16)] = tfvec
                return j + 1, off + 16

            lax.while_loop(lambda carry: carry[1] < cap, fill, (0, ngt))

            pltpu.sync_copy(si_v.at[pl.ds(0, cap)], idx_hbm.at[e])
            pltpu.sync_copy(sv_v.at[pl.ds(0, cap)], sc_hbm.at[e])

        plsc.subcore_barrier()
        eg = c * epc + s // 4
        part = s % 4
        nslice = 2 * _GCHUNK
        pltpu.sync_copy(idx_hbm.at[eg, pl.ds(part * nslice, nslice)], ic_v)

        def gath(g, carry):
            pltpu.async_copy(x_hbm.at[ic_v.at[pl.ds(g * _GCHUNK, _GCHUNK)]],
                             rows_v, sem).wait()
            base = eg * cap + part * nslice + g * _GCHUNK
            pltpu.sync_copy(rows_v, xs_hbm.at[pl.ds(base, _GCHUNK)])
            return carry

        lax.fori_loop(0, 2, gath, 0)

    return k(probs_t, thresh16, x2)


def _expert_body(xs_ref, ekw_ref, ekb_ref, evw_ref, evb_ref, sc_ref, out_ref):
    xs_bf = xs_ref[...].astype(jnp.bfloat16)
    kw_bf = ekw_ref[0].astype(jnp.bfloat16)
    h = jnp.dot(xs_bf, kw_bf, preferred_element_type=jnp.float32)
    h = jax.nn.gelu(h + ekb_ref[0])
    vw_bf = evw_ref[0].astype(jnp.bfloat16)
    o = jnp.dot(h.astype(jnp.bfloat16), vw_bf, preferred_element_type=jnp.float32)
    o = o + evb_ref[0]
    out_ref[...] = o * sc_ref[0, 0][:, None]


def _routed_ffn(xs, ek_w, ek_b, ev_w, ev_b, scores, cap):
    E = NUM_EXPERTS
    return pl.pallas_call(
        _expert_body,
        grid=(E,),
        in_specs=[
            pl.BlockSpec((cap, D_MODEL), lambda e: (e, 0)),
            pl.BlockSpec((1, D_MODEL, HIDDEN), lambda e: (e, 0, 0)),
            pl.BlockSpec((1, 1, HIDDEN), lambda e: (e, 0, 0)),
            pl.BlockSpec((1, HIDDEN, D_MODEL), lambda e: (e, 0, 0)),
            pl.BlockSpec((1, 1, D_MODEL), lambda e: (e, 0, 0)),
            pl.BlockSpec((1, 1, cap), lambda e: (e, 0, 0)),
        ],
        out_specs=pl.BlockSpec((cap, D_MODEL), lambda e: (e, 0)),
        out_shape=jax.ShapeDtypeStruct((E * cap, D_MODEL), jnp.float32),
    )(xs, ek_w, ek_b.reshape(E, 1, HIDDEN), ev_w,
      ev_b.reshape(E, 1, D_MODEL), scores.reshape(E, 1, cap))


def _shared_body(x_ref, skw_ref, skb_ref, svw_ref, svb_ref, out_ref):
    x_bf = x_ref[...].astype(jnp.bfloat16)
    kw_bf = skw_ref[...].astype(jnp.bfloat16)
    h = jnp.dot(x_bf, kw_bf, preferred_element_type=jnp.float32)
    h = jax.nn.gelu(h + skb_ref[...])
    vw_bf = svw_ref[...].astype(jnp.bfloat16)
    o = jnp.dot(h.astype(jnp.bfloat16), vw_bf, preferred_element_type=jnp.float32)
    out_ref[...] = o + svb_ref[...]


def _shared_ffn(x2, sk_w, sk_b, sv_w, sv_b, blk):
    T = x2.shape[0]
    return pl.pallas_call(
        _shared_body,
        grid=(T // blk,),
        in_specs=[
            pl.BlockSpec((blk, D_MODEL), lambda i: (i, 0)),
            pl.BlockSpec((D_MODEL, HIDDEN), lambda i: (0, 0)),
            pl.BlockSpec((1, HIDDEN), lambda i: (0, 0)),
            pl.BlockSpec((HIDDEN, D_MODEL), lambda i: (0, 0)),
            pl.BlockSpec((1, D_MODEL), lambda i: (0, 0)),
        ],
        out_specs=pl.BlockSpec((blk, D_MODEL), lambda i: (i, 0)),
        out_shape=jax.ShapeDtypeStruct((T, D_MODEL), jnp.float32),
    )(x2, sk_w, sk_b.reshape(1, HIDDEN), sv_w, sv_b.reshape(1, D_MODEL))


def kernel(x, gate_w, gate_b, temperature, ek_w, ek_b, ev_w, ev_b,
           sk_w, sk_b, sv_w, sv_b):
    B, S, D = x.shape
    T = B * S
    cap = max(8, T // NUM_EXPERTS)
    x2 = x.reshape(T, D)

    probs_t, thr = _router(x2, gate_w, gate_b, temperature, cap)
    thresh16 = jnp.pad(thr[:, 0], (0, 16 - NUM_EXPERTS))

    xs, token_idx, scores = _topk_gather_sc(probs_t, thresh16, x2, cap)

    slots = _routed_ffn(xs, ek_w, ek_b, ev_w, ev_b, scores, cap)
    shared = _shared_ffn(x2, sk_w, sk_b, sv_w, sv_b, blk=512)

    out = shared.at[token_idx.reshape(-1)].add(slots)
    return out.reshape(B, S, D)


# speedup vs baseline: 1.9089x; 1.0180x over previous
import dataclasses
import functools

import jax
import jax.numpy as jnp
from jax import lax
from jax.experimental import pallas as pl
from jax.experimental.pallas import tpu as pltpu
from jax.experimental.pallas import tpu_sc as plsc

D_MODEL = 1024
HIDDEN = 2048
NUM_EXPERTS = 8
ONE_BITS = 0x3F800001


def _router_body(cap, x_ref, gw_ref, gb_ref, temp_ref, probs_ref, thr_ref):
    x_bf = x_ref[...].astype(jnp.bfloat16)
    gw_bf = gw_ref[...].astype(jnp.bfloat16)
    logits = lax.dot_general(gw_bf, x_bf, (((0,), (1,)), ((), ())),
                             preferred_element_type=jnp.float32)
    logits = logits + gb_ref[...]
    safe_temp = jnp.maximum(temp_ref[0, 0], 0.1)
    logits = logits / safe_temp
    m = jnp.max(logits, axis=0, keepdims=True)
    ex = jnp.exp(logits - m)
    probs = ex / jnp.sum(ex, axis=0, keepdims=True)
    probs_ref[...] = probs

    bits = lax.bitcast_convert_type(probs, jnp.int32)

    def it(_, lohi):
        lo, hi = lohi
        mid = lo + (hi - lo) // 2
        cnt = jnp.sum((bits >= mid).astype(jnp.int32), axis=1, keepdims=True)
        ge = cnt >= cap
        return jnp.where(ge, mid, lo), jnp.where(ge, hi, mid)

    lo0 = jnp.zeros((NUM_EXPERTS, 1), jnp.int32)
    hi0 = jnp.full((NUM_EXPERTS, 1), ONE_BITS, jnp.int32)
    lo, _ = lax.fori_loop(0, 31, it, (lo0, hi0))
    thr_ref[...] = jnp.broadcast_to(lo, (NUM_EXPERTS, 128))


def _router(x2, gate_w, gate_b, temperature, cap):
    T = x2.shape[0]
    return pl.pallas_call(
        functools.partial(_router_body, cap),
        out_shape=[jax.ShapeDtypeStruct((NUM_EXPERTS, T), jnp.float32),
                   jax.ShapeDtypeStruct((NUM_EXPERTS, 128), jnp.int32)],
    )(x2, gate_w, gate_b.reshape(NUM_EXPERTS, 1), temperature.reshape(1, 1))


_GCHUNK = 64


def _topk_gather_sc(probs_t, thresh16, x2, cap):
    T, D = x2.shape
    E = NUM_EXPERTS
    epc = E // 2
    mesh = plsc.VectorSubcoreMesh(core_axis_name="c", subcore_axis_name="s")

    cp = pltpu.CompilerParams()
    if "needs_layout_passes" in pltpu.CompilerParams.__dataclass_fields__:
        cp = dataclasses.replace(cp, needs_layout_passes=False)

    @functools.partial(
        pl.kernel,
        compiler_params=cp,
        out_type=[jax.ShapeDtypeStruct((E * cap, D), jnp.float32),
                  jax.ShapeDtypeStruct((E, cap), jnp.int32),
                  jax.ShapeDtypeStruct((E, cap), jnp.float32)],
        mesh=mesh,
        scratch_types=[
            pltpu.VMEM((T,), jnp.float32),
            pltpu.VMEM((cap + 32,), jnp.int32),
            pltpu.VMEM((cap + 32,), jnp.float32),
            pltpu.VMEM((cap + 32,), jnp.int32),
            pltpu.VMEM((16,), jnp.int32),
            pltpu.VMEM((2 * _GCHUNK,), jnp.int32),
            pltpu.VMEM((_GCHUNK, D), jnp.float32),
            pltpu.SemaphoreType.DMA,
        ],
    )
    def k(probs_hbm, th_hbm, x_hbm, xs_hbm, idx_hbm, sc_hbm,
          pf_v, si_v, sv_v, se_v, th_v, ic_v, rows_v, sem):
        c = lax.axis_index("c")
        s = lax.axis_index("s")

        @pl.when(s < epc)
        def _():
            e = c * epc + s
            pltpu.sync_copy(probs_hbm.at[e], pf_v)
            pltpu.sync_copy(th_hbm, th_v)
            evec = jnp.full((16,), e, jnp.int32)
            tvec = plsc.load_gather(th_v, [evec])
            tfvec = plsc.bitcast(tvec, jnp.float32)
            lanes = jnp.arange(16, dtype=jnp.int32)

            def scan_chunk(j4, carry):
                off, eoff = carry
                for u in range(4):
                    j = j4 * 4 + u
                    pf = pf_v[pl.ds(j * 16, 16)]
                    pb = plsc.bitcast(pf, jnp.int32)
                    iv = lanes + j * 16
                    mg = pb > tvec
                    cg = jnp.max(plsc.all_reduce_population_count(mg))
                    plsc.store_compressed(si_v.at[pl.ds(off, 16)], iv, mask=mg)
                    plsc.store_compressed(sv_v.at[pl.ds(off, 16)], pf, mask=mg)
                    off = off + cg
                    me = pb == tvec
                    ce = jnp.max(plsc.all_reduce_population_count(me))

                    @pl.when(eoff < cap)
                    def _():
                        plsc.store_compressed(se_v.at[pl.ds(eoff, 16)], iv,
                                              mask=me)

                    eoff = eoff + jnp.where(eoff < cap, ce, 0)
                return off, eoff

            ngt, _ = lax.fori_loop(0, T // 64, scan_chunk, (0, 0))

            def fill(carry):
                j, off = carry
                si_v[pl.ds(off, 16)] = se_v[pl.ds(j * 16, 16)]
                sv_v[pl.ds(off, 16)] = tfvec
                return j + 1, off + 16

            lax.while_loop(lambda carry: carry[1] < cap, fill, (0, ngt))

            pltpu.sync_copy(si_v.at[pl.ds(0, cap)], idx_hbm.at[e])
            pltpu.sync_copy(sv_v.at[pl.ds(0, cap)], sc_hbm.at[e])

        plsc.subcore_barrier()
        eg = c * epc + s // 4
        part = s % 4
        nslice = 2 * _GCHUNK
        pltpu.sync_copy(idx_hbm.at[eg, pl.ds(part * nslice, nslice)], ic_v)

        def gath(g, carry):
            pltpu.async_copy(x_hbm.at[ic_v.at[pl.ds(g * _GCHUNK, _GCHUNK)]],
                             rows_v, sem).wait()
            base = eg * cap + part * nslice + g * _GCHUNK
            pltpu.sync_copy(rows_v, xs_hbm.at[pl.ds(base, _GCHUNK)])
            return carry

        lax.fori_loop(0, 2, gath, 0)

    return k(probs_t, thresh16, x2)


def _expert_body(xs_ref, ekw_ref, ekb_ref, evw_ref, evb_ref, sc_ref, out_ref):
    xs_bf = xs_ref[...].astype(jnp.bfloat16)
    kw_bf = ekw_ref[0].astype(jnp.bfloat16)
    h = jnp.dot(xs_bf, kw_bf, preferred_element_type=jnp.float32)
    h = jax.nn.gelu(h + ekb_ref[0])
    vw_bf = evw_ref[0].astype(jnp.bfloat16)
    o = jnp.dot(h.astype(jnp.bfloat16), vw_bf, preferred_element_type=jnp.float32)
    o = o + evb_ref[0]
    out_ref[...] = (o * sc_ref[0, 0][:, None]).astype(jnp.bfloat16)


def _routed_ffn(xs, ek_w, ek_b, ev_w, ev_b, scores, cap):
    E = NUM_EXPERTS
    return pl.pallas_call(
        _expert_body,
        grid=(E,),
        in_specs=[
            pl.BlockSpec((cap, D_MODEL), lambda e: (e, 0)),
            pl.BlockSpec((1, D_MODEL, HIDDEN), lambda e: (e, 0, 0)),
            pl.BlockSpec((1, 1, HIDDEN), lambda e: (e, 0, 0)),
            pl.BlockSpec((1, HIDDEN, D_MODEL), lambda e: (e, 0, 0)),
            pl.BlockSpec((1, 1, D_MODEL), lambda e: (e, 0, 0)),
            pl.BlockSpec((1, 1, cap), lambda e: (e, 0, 0)),
        ],
        out_specs=pl.BlockSpec((cap, D_MODEL), lambda e: (e, 0)),
        out_shape=jax.ShapeDtypeStruct((E * cap, D_MODEL), jnp.bfloat16),
    )(xs, ek_w, ek_b.reshape(E, 1, HIDDEN), ev_w,
      ev_b.reshape(E, 1, D_MODEL), scores.reshape(E, 1, cap))


def _shared_body(x_ref, skw_ref, skb_ref, svw_ref, svb_ref, out_ref):
    x_bf = x_ref[...].astype(jnp.bfloat16)
    kw_bf = skw_ref[...].astype(jnp.bfloat16)
    h = jnp.dot(x_bf, kw_bf, preferred_element_type=jnp.float32)
    h = jax.nn.gelu(h + skb_ref[...])
    vw_bf = svw_ref[...].astype(jnp.bfloat16)
    o = jnp.dot(h.astype(jnp.bfloat16), vw_bf, preferred_element_type=jnp.float32)
    out_ref[...] = o + svb_ref[...]


def _shared_ffn(x2, sk_w, sk_b, sv_w, sv_b, blk):
    T = x2.shape[0]
    return pl.pallas_call(
        _shared_body,
        grid=(T // blk,),
        in_specs=[
            pl.BlockSpec((blk, D_MODEL), lambda i: (i, 0)),
            pl.BlockSpec((D_MODEL, HIDDEN), lambda i: (0, 0)),
            pl.BlockSpec((1, HIDDEN), lambda i: (0, 0)),
            pl.BlockSpec((HIDDEN, D_MODEL), lambda i: (0, 0)),
            pl.BlockSpec((1, D_MODEL), lambda i: (0, 0)),
        ],
        out_specs=pl.BlockSpec((blk, D_MODEL), lambda i: (i, 0)),
        out_shape=jax.ShapeDtypeStruct((T, D_MODEL), jnp.float32),
    )(x2, sk_w, sk_b.reshape(1, HIDDEN), sv_w, sv_b.reshape(1, D_MODEL))


def kernel(x, gate_w, gate_b, temperature, ek_w, ek_b, ev_w, ev_b,
           sk_w, sk_b, sv_w, sv_b):
    B, S, D = x.shape
    T = B * S
    cap = max(8, T // NUM_EXPERTS)
    x2 = x.reshape(T, D)

    probs_t, thr = _router(x2, gate_w, gate_b, temperature, cap)
    thresh16 = jnp.pad(thr[:, 0], (0, 16 - NUM_EXPERTS))

    xs, token_idx, scores = _topk_gather_sc(probs_t, thresh16, x2, cap)

    slots = _routed_ffn(xs, ek_w, ek_b, ev_w, ev_b, scores, cap)
    shared = _shared_ffn(x2, sk_w, sk_b, sv_w, sv_b, blk=512)

    out = shared.at[token_idx.reshape(-1)].add(slots)
    return out.reshape(B, S, D)


# speedup vs baseline: 1.9094x; 1.0003x over previous
import dataclasses
import functools

import jax
import jax.numpy as jnp
from jax import lax
from jax.experimental import pallas as pl
from jax.experimental.pallas import tpu as pltpu
from jax.experimental.pallas import tpu_sc as plsc

D_MODEL = 1024
HIDDEN = 2048
NUM_EXPERTS = 8
ONE_BITS = 0x3F800001


def _router_body(cap, x_ref, gw_ref, gb_ref, temp_ref, probs_ref, thr_ref):
    x_bf = x_ref[...].astype(jnp.bfloat16)
    gw_bf = gw_ref[...].astype(jnp.bfloat16)
    logits = lax.dot_general(gw_bf, x_bf, (((0,), (1,)), ((), ())),
                             preferred_element_type=jnp.float32)
    logits = logits + gb_ref[...]
    safe_temp = jnp.maximum(temp_ref[0, 0], 0.1)
    logits = logits / safe_temp
    m = jnp.max(logits, axis=0, keepdims=True)
    ex = jnp.exp(logits - m)
    probs = ex / jnp.sum(ex, axis=0, keepdims=True)
    probs_ref[...] = probs

    bits = lax.bitcast_convert_type(probs, jnp.int32)

    def it(_, lohi):
        lo, hi = lohi
        mid = lo + (hi - lo) // 2
        cnt = jnp.sum((bits >= mid).astype(jnp.int32), axis=1, keepdims=True)
        ge = cnt >= cap
        return jnp.where(ge, mid, lo), jnp.where(ge, hi, mid)

    lo0 = jnp.zeros((NUM_EXPERTS, 1), jnp.int32)
    hi0 = jnp.full((NUM_EXPERTS, 1), ONE_BITS, jnp.int32)
    lo, _ = lax.fori_loop(0, 31, it, (lo0, hi0))
    thr_ref[...] = jnp.broadcast_to(lo, (NUM_EXPERTS, 128))


def _router(x2, gate_w, gate_b, temperature, cap):
    T = x2.shape[0]
    return pl.pallas_call(
        functools.partial(_router_body, cap),
        out_shape=[jax.ShapeDtypeStruct((NUM_EXPERTS, T), jnp.float32),
                   jax.ShapeDtypeStruct((NUM_EXPERTS, 128), jnp.int32)],
    )(x2, gate_w, gate_b.reshape(NUM_EXPERTS, 1), temperature.reshape(1, 1))


_GCHUNK = 64


def _topk_gather_sc(probs_t, thresh16, x2, cap):
    T, D = x2.shape
    E = NUM_EXPERTS
    epc = E // 2
    mesh = plsc.VectorSubcoreMesh(core_axis_name="c", subcore_axis_name="s")

    cp = pltpu.CompilerParams()
    if "needs_layout_passes" in pltpu.CompilerParams.__dataclass_fields__:
        cp = dataclasses.replace(cp, needs_layout_passes=False)

    @functools.partial(
        pl.kernel,
        compiler_params=cp,
        out_type=[jax.ShapeDtypeStruct((E * cap, D), jnp.float32),
                  jax.ShapeDtypeStruct((E, cap), jnp.int32),
                  jax.ShapeDtypeStruct((E, cap), jnp.float32)],
        mesh=mesh,
        scratch_types=[
            pltpu.VMEM((T,), jnp.float32),
            pltpu.VMEM((cap + 32,), jnp.int32),
            pltpu.VMEM((cap + 32,), jnp.float32),
            pltpu.VMEM((cap + 32,), jnp.int32),
            pltpu.VMEM((16,), jnp.int32),
            pltpu.VMEM((2 * _GCHUNK,), jnp.int32),
            pltpu.VMEM((_GCHUNK // 2, D), jnp.float32),
            pltpu.VMEM((_GCHUNK // 2, D), jnp.float32),
            pltpu.SemaphoreType.DMA,
            pltpu.SemaphoreType.DMA,
        ],
    )
    def k(probs_hbm, th_hbm, x_hbm, xs_hbm, idx_hbm, sc_hbm,
          pf_v, si_v, sv_v, se_v, th_v, ic_v, rows_a, rows_b, sem, wsem):
        c = lax.axis_index("c")
        s = lax.axis_index("s")

        @pl.when(s < epc)
        def _():
            e = c * epc + s
            pltpu.sync_copy(probs_hbm.at[e], pf_v)
            pltpu.sync_copy(th_hbm, th_v)
            evec = jnp.full((16,), e, jnp.int32)
            tvec = plsc.load_gather(th_v, [evec])
            tfvec = plsc.bitcast(tvec, jnp.float32)
            lanes = jnp.arange(16, dtype=jnp.int32)

            def scan_chunk(j4, carry):
                off, eoff = carry
                for u in range(4):
                    j = j4 * 4 + u
                    pf = pf_v[pl.ds(j * 16, 16)]
                    pb = plsc.bitcast(pf, jnp.int32)
                    iv = lanes + j * 16
                    mg = pb > tvec
                    cg = jnp.max(plsc.all_reduce_population_count(mg))
                    plsc.store_compressed(si_v.at[pl.ds(off, 16)], iv, mask=mg)
                    plsc.store_compressed(sv_v.at[pl.ds(off, 16)], pf, mask=mg)
                    off = off + cg
                    me = pb == tvec
                    ce = jnp.max(plsc.all_reduce_population_count(me))

                    @pl.when(eoff < cap)
                    def _():
                        plsc.store_compressed(se_v.at[pl.ds(eoff, 16)], iv,
                                              mask=me)

                    eoff = eoff + jnp.where(eoff < cap, ce, 0)
                return off, eoff

            ngt, _ = lax.fori_loop(0, T // 64, scan_chunk, (0, 0))

            def fill(carry):
                j, off = carry
                si_v[pl.ds(off, 16)] = se_v[pl.ds(j * 16, 16)]
                sv_v[pl.ds(off, 16)] = tfvec
                return j + 1, off + 16

            lax.while_loop(lambda carry: carry[1] < cap, fill, (0, ngt))

            pltpu.sync_copy(si_v.at[pl.ds(0, cap)], idx_hbm.at[e])
            pltpu.sync_copy(sv_v.at[pl.ds(0, cap)], sc_hbm.at[e])

        plsc.subcore_barrier()
        eg = c * epc + s // 4
        part = s % 4
        nslice = 2 * _GCHUNK
        pltpu.sync_copy(idx_hbm.at[eg, pl.ds(part * nslice, nslice)], ic_v)

        base = eg * cap + part * nslice
        gc = _GCHUNK // 2
        bufs = (rows_a, rows_b)
        wbs = []
        for g in range(4):
            buf = bufs[g % 2]
            if g >= 2:
                wbs[g - 2].wait()
            pltpu.async_copy(x_hbm.at[ic_v.at[pl.ds(g * gc, gc)]],
                             buf, sem).wait()
            wbs.append(pltpu.async_copy(
                buf, xs_hbm.at[pl.ds(base + g * gc, gc)], wsem))
        wbs[2].wait()
        wbs[3].wait()

    return k(probs_t, thresh16, x2)


def _expert_body(xs_ref, ekw_ref, ekb_ref, evw_ref, evb_ref, sc_ref, out_ref):
    xs_bf = xs_ref[...].astype(jnp.bfloat16)
    kw_bf = ekw_ref[0].astype(jnp.bfloat16)
    h = jnp.dot(xs_bf, kw_bf, preferred_element_type=jnp.float32)
    h = jax.nn.gelu(h + ekb_ref[0])
    vw_bf = evw_ref[0].astype(jnp.bfloat16)
    o = jnp.dot(h.astype(jnp.bfloat16), vw_bf, preferred_element_type=jnp.float32)
    o = o + evb_ref[0]
    out_ref[...] = (o * sc_ref[0, 0][:, None]).astype(jnp.bfloat16)


def _routed_ffn(xs, ek_w, ek_b, ev_w, ev_b, scores, cap):
    E = NUM_EXPERTS
    return pl.pallas_call(
        _expert_body,
        grid=(E,),
        in_specs=[
            pl.BlockSpec((cap, D_MODEL), lambda e: (e, 0)),
            pl.BlockSpec((1, D_MODEL, HIDDEN), lambda e: (e, 0, 0)),
            pl.BlockSpec((1, 1, HIDDEN), lambda e: (e, 0, 0)),
            pl.BlockSpec((1, HIDDEN, D_MODEL), lambda e: (e, 0, 0)),
            pl.BlockSpec((1, 1, D_MODEL), lambda e: (e, 0, 0)),
            pl.BlockSpec((1, 1, cap), lambda e: (e, 0, 0)),
        ],
        out_specs=pl.BlockSpec((cap, D_MODEL), lambda e: (e, 0)),
        out_shape=jax.ShapeDtypeStruct((E * cap, D_MODEL), jnp.bfloat16),
    )(xs, ek_w, ek_b.reshape(E, 1, HIDDEN), ev_w,
      ev_b.reshape(E, 1, D_MODEL), scores.reshape(E, 1, cap))


def _shared_body(x_ref, skw_ref, skb_ref, svw_ref, svb_ref, out_ref):
    x_bf = x_ref[...].astype(jnp.bfloat16)
    kw_bf = skw_ref[...].astype(jnp.bfloat16)
    h = jnp.dot(x_bf, kw_bf, preferred_element_type=jnp.float32)
    h = jax.nn.gelu(h + skb_ref[...])
    vw_bf = svw_ref[...].astype(jnp.bfloat16)
    o = jnp.dot(h.astype(jnp.bfloat16), vw_bf, preferred_element_type=jnp.float32)
    out_ref[...] = o + svb_ref[...]


def _shared_ffn(x2, sk_w, sk_b, sv_w, sv_b, blk):
    T = x2.shape[0]
    return pl.pallas_call(
        _shared_body,
        grid=(T // blk,),
        in_specs=[
            pl.BlockSpec((blk, D_MODEL), lambda i: (i, 0)),
            pl.BlockSpec((D_MODEL, HIDDEN), lambda i: (0, 0)),
            pl.BlockSpec((1, HIDDEN), lambda i: (0, 0)),
            pl.BlockSpec((HIDDEN, D_MODEL), lambda i: (0, 0)),
            pl.BlockSpec((1, D_MODEL), lambda i: (0, 0)),
        ],
        out_specs=pl.BlockSpec((blk, D_MODEL), lambda i: (i, 0)),
        out_shape=jax.ShapeDtypeStruct((T, D_MODEL), jnp.float32),
    )(x2, sk_w, sk_b.reshape(1, HIDDEN), sv_w, sv_b.reshape(1, D_MODEL))


def kernel(x, gate_w, gate_b, temperature, ek_w, ek_b, ev_w, ev_b,
           sk_w, sk_b, sv_w, sv_b):
    B, S, D = x.shape
    T = B * S
    cap = max(8, T // NUM_EXPERTS)
    x2 = x.reshape(T, D)

    probs_t, thr = _router(x2, gate_w, gate_b, temperature, cap)
    thresh16 = jnp.pad(thr[:, 0], (0, 16 - NUM_EXPERTS))

    xs, token_idx, scores = _topk_gather_sc(probs_t, thresh16, x2, cap)

    slots = _routed_ffn(xs, ek_w, ek_b, ev_w, ev_b, scores, cap)
    shared = _shared_ffn(x2, sk_w, sk_b, sv_w, sv_b, blk=512)

    out = shared.at[token_idx.reshape(-1)].add(slots)
    return out.reshape(B, S, D)


# speedup vs baseline: 1.9221x; 1.0067x over previous
import dataclasses
import functools

import jax
import jax.numpy as jnp
from jax import lax
from jax.experimental import pallas as pl
from jax.experimental.pallas import tpu as pltpu
from jax.experimental.pallas import tpu_sc as plsc

D_MODEL = 1024
HIDDEN = 2048
NUM_EXPERTS = 8
ONE_BITS = 0x3F800001


def _router_body(cap, x_ref, gw_ref, gb_ref, temp_ref, probs_ref, thr_ref):
    x_bf = x_ref[...].astype(jnp.bfloat16)
    gw_bf = gw_ref[...].astype(jnp.bfloat16)
    logits = lax.dot_general(gw_bf, x_bf, (((0,), (1,)), ((), ())),
                             preferred_element_type=jnp.float32)
    logits = logits + gb_ref[...]
    safe_temp = jnp.maximum(temp_ref[0, 0], 0.1)
    logits = logits / safe_temp
    m = jnp.max(logits, axis=0, keepdims=True)
    ex = jnp.exp(logits - m)
    probs = ex / jnp.sum(ex, axis=0, keepdims=True)
    probs_ref[...] = probs

    bits = lax.bitcast_convert_type(probs, jnp.int32)

    def it(_, lohi):
        lo, hi = lohi
        mid = lo + (hi - lo) // 2
        cnt = jnp.sum((bits >= mid).astype(jnp.int32), axis=1, keepdims=True)
        ge = cnt >= cap
        return jnp.where(ge, mid, lo), jnp.where(ge, hi, mid)

    lo0 = jnp.zeros((NUM_EXPERTS, 1), jnp.int32)
    hi0 = jnp.full((NUM_EXPERTS, 1), ONE_BITS, jnp.int32)
    lo, _ = lax.fori_loop(0, 31, it, (lo0, hi0))
    thr_ref[...] = jnp.broadcast_to(lo, (NUM_EXPERTS, 128))


def _router(x2, gate_w, gate_b, temperature, cap):
    T = x2.shape[0]
    return pl.pallas_call(
        functools.partial(_router_body, cap),
        out_shape=[jax.ShapeDtypeStruct((NUM_EXPERTS, T), jnp.float32),
                   jax.ShapeDtypeStruct((NUM_EXPERTS, 128), jnp.int32)],
    )(x2, gate_w, gate_b.reshape(NUM_EXPERTS, 1), temperature.reshape(1, 1))


_GCHUNK = 64


def _topk_gather_sc(probs_t, thr, x2, cap):
    T, D = x2.shape
    E = NUM_EXPERTS
    epc = E // 2
    mesh = plsc.VectorSubcoreMesh(core_axis_name="c", subcore_axis_name="s")

    cp = pltpu.CompilerParams()
    if "needs_layout_passes" in pltpu.CompilerParams.__dataclass_fields__:
        cp = dataclasses.replace(cp, needs_layout_passes=False)

    @functools.partial(
        pl.kernel,
        compiler_params=cp,
        out_type=[jax.ShapeDtypeStruct((E * cap, D), jnp.float32),
                  jax.ShapeDtypeStruct((E, cap), jnp.int32),
                  jax.ShapeDtypeStruct((E, cap), jnp.float32)],
        mesh=mesh,
        scratch_types=[
            pltpu.VMEM((T,), jnp.float32),
            pltpu.VMEM((cap + 32,), jnp.int32),
            pltpu.VMEM((cap + 32,), jnp.float32),
            pltpu.VMEM((cap + 32,), jnp.int32),
            pltpu.VMEM((16,), jnp.int32),
            pltpu.VMEM((2 * _GCHUNK,), jnp.int32),
            pltpu.VMEM((_GCHUNK // 2, D), jnp.float32),
            pltpu.VMEM((_GCHUNK // 2, D), jnp.float32),
            pltpu.SemaphoreType.DMA,
            pltpu.SemaphoreType.DMA,
        ],
    )
    def k(probs_hbm, th_hbm, x_hbm, xs_hbm, idx_hbm, sc_hbm,
          pf_v, si_v, sv_v, se_v, th_v, ic_v, rows_a, rows_b, sem, wsem):
        c = lax.axis_index("c")
        s = lax.axis_index("s")

        @pl.when(s < epc)
        def _():
            e = c * epc + s
            pltpu.sync_copy(probs_hbm.at[e], pf_v)
            pltpu.sync_copy(th_hbm.at[e, pl.ds(0, 16)], th_v)
            zvec = jnp.zeros((16,), jnp.int32)
            tvec = plsc.load_gather(th_v, [zvec])
            tfvec = plsc.bitcast(tvec, jnp.float32)
            lanes = jnp.arange(16, dtype=jnp.int32)

            def scan_chunk(j4, carry):
                off, eoff = carry
                for u in range(4):
                    j = j4 * 4 + u
                    pf = pf_v[pl.ds(j * 16, 16)]
                    pb = plsc.bitcast(pf, jnp.int32)
                    iv = lanes + j * 16
                    mg = pb > tvec
                    cg = jnp.max(plsc.all_reduce_population_count(mg))
                    plsc.store_compressed(si_v.at[pl.ds(off, 16)], iv, mask=mg)
                    plsc.store_compressed(sv_v.at[pl.ds(off, 16)], pf, mask=mg)
                    off = off + cg
                    me = pb == tvec
                    ce = jnp.max(plsc.all_reduce_population_count(me))

                    @pl.when(eoff < cap)
                    def _():
                        plsc.store_compressed(se_v.at[pl.ds(eoff, 16)], iv,
                                              mask=me)

                    eoff = eoff + jnp.where(eoff < cap, ce, 0)
                return off, eoff

            ngt, _ = lax.fori_loop(0, T // 64, scan_chunk, (0, 0))

            def fill(carry):
                j, off = carry
                si_v[pl.ds(off, 16)] = se_v[pl.ds(j * 16, 16)]
                sv_v[pl.ds(off, 16)] = tfvec
                return j + 1, off + 16

            lax.while_loop(lambda carry: carry[1] < cap, fill, (0, ngt))

            pltpu.sync_copy(si_v.at[pl.ds(0, cap)], idx_hbm.at[e])
            pltpu.sync_copy(sv_v.at[pl.ds(0, cap)], sc_hbm.at[e])

        plsc.subcore_barrier()
        eg = c * epc + s // 4
        part = s % 4
        nslice = 2 * _GCHUNK
        pltpu.sync_copy(idx_hbm.at[eg, pl.ds(part * nslice, nslice)], ic_v)

        base = eg * cap + part * nslice
        gc = _GCHUNK // 2
        bufs = (rows_a, rows_b)
        wbs = []
        for g in range(4):
            buf = bufs[g % 2]
            if g >= 2:
                wbs[g - 2].wait()
            pltpu.async_copy(x_hbm.at[ic_v.at[pl.ds(g * gc, gc)]],
                             buf, sem).wait()
            wbs.append(pltpu.async_copy(
                buf, xs_hbm.at[pl.ds(base + g * gc, gc)], wsem))
        wbs[2].wait()
        wbs[3].wait()

    return k(probs_t, thr, x2)


def _expert_body(xs_ref, ekw_ref, ekb_ref, evw_ref, evb_ref, sc_ref, out_ref):
    xs_bf = xs_ref[...].astype(jnp.bfloat16)
    kw_bf = ekw_ref[0].astype(jnp.bfloat16)
    h = jnp.dot(xs_bf, kw_bf, preferred_element_type=jnp.float32)
    h = jax.nn.gelu(h + ekb_ref[0])
    vw_bf = evw_ref[0].astype(jnp.bfloat16)
    o = jnp.dot(h.astype(jnp.bfloat16), vw_bf, preferred_element_type=jnp.float32)
    o = o + evb_ref[0]
    out_ref[...] = (o * sc_ref[0, 0][:, None]).astype(jnp.bfloat16)


def _routed_ffn(xs, ek_w, ek_b, ev_w, ev_b, scores, cap):
    E = NUM_EXPERTS
    return pl.pallas_call(
        _expert_body,
        grid=(E,),
        in_specs=[
            pl.BlockSpec((cap, D_MODEL), lambda e: (e, 0)),
            pl.BlockSpec((1, D_MODEL, HIDDEN), lambda e: (e, 0, 0)),
            pl.BlockSpec((1, 1, HIDDEN), lambda e: (e, 0, 0)),
            pl.BlockSpec((1, HIDDEN, D_MODEL), lambda e: (e, 0, 0)),
            pl.BlockSpec((1, 1, D_MODEL), lambda e: (e, 0, 0)),
            pl.BlockSpec((1, 1, cap), lambda e: (e, 0, 0)),
        ],
        out_specs=pl.BlockSpec((cap, D_MODEL), lambda e: (e, 0)),
        out_shape=jax.ShapeDtypeStruct((E * cap, D_MODEL), jnp.bfloat16),
    )(xs, ek_w, ek_b.reshape(E, 1, HIDDEN), ev_w,
      ev_b.reshape(E, 1, D_MODEL), scores.reshape(E, 1, cap))


def _shared_body(x_ref, skw_ref, skb_ref, svw_ref, svb_ref, out_ref):
    x_bf = x_ref[...].astype(jnp.bfloat16)
    kw_bf = skw_ref[...].astype(jnp.bfloat16)
    h = jnp.dot(x_bf, kw_bf, preferred_element_type=jnp.float32)
    h = jax.nn.gelu(h + skb_ref[...])
    vw_bf = svw_ref[...].astype(jnp.bfloat16)
    o = jnp.dot(h.astype(jnp.bfloat16), vw_bf, preferred_element_type=jnp.float32)
    out_ref[...] = o + svb_ref[...]


def _shared_ffn(x2, sk_w, sk_b, sv_w, sv_b, blk):
    T = x2.shape[0]
    return pl.pallas_call(
        _shared_body,
        grid=(T // blk,),
        in_specs=[
            pl.BlockSpec((blk, D_MODEL), lambda i: (i, 0)),
            pl.BlockSpec((D_MODEL, HIDDEN), lambda i: (0, 0)),
            pl.BlockSpec((1, HIDDEN), lambda i: (0, 0)),
            pl.BlockSpec((HIDDEN, D_MODEL), lambda i: (0, 0)),
            pl.BlockSpec((1, D_MODEL), lambda i: (0, 0)),
        ],
        out_specs=pl.BlockSpec((blk, D_MODEL), lambda i: (i, 0)),
        out_shape=jax.ShapeDtypeStruct((T, D_MODEL), jnp.float32),
    )(x2, sk_w, sk_b.reshape(1, HIDDEN), sv_w, sv_b.reshape(1, D_MODEL))


def kernel(x, gate_w, gate_b, temperature, ek_w, ek_b, ev_w, ev_b,
           sk_w, sk_b, sv_w, sv_b):
    B, S, D = x.shape
    T = B * S
    cap = max(8, T // NUM_EXPERTS)
    x2 = x.reshape(T, D)

    probs_t, thr = _router(x2, gate_w, gate_b, temperature, cap)

    xs, token_idx, scores = _topk_gather_sc(probs_t, thr, x2, cap)

    slots = _routed_ffn(xs, ek_w, ek_b, ev_w, ev_b, scores, cap)
    shared = _shared_ffn(x2, sk_w, sk_b, sv_w, sv_b, blk=512)

    out = shared.at[token_idx.reshape(-1)].add(slots)
    return out.reshape(B, S, D)


# speedup vs baseline: 1.9227x; 1.0003x over previous
import dataclasses
import functools

import jax
import jax.numpy as jnp
from jax import lax
from jax.experimental import pallas as pl
from jax.experimental.pallas import tpu as pltpu
from jax.experimental.pallas import tpu_sc as plsc

D_MODEL = 1024
HIDDEN = 2048
NUM_EXPERTS = 8
ONE_BITS = 0x3F800001


def _router_body(cap, x_ref, gw_ref, gb_ref, temp_ref, probs_ref, thr_ref):
    x_bf = x_ref[...].astype(jnp.bfloat16)
    gw_bf = gw_ref[...].astype(jnp.bfloat16)
    logits = lax.dot_general(gw_bf, x_bf, (((0,), (1,)), ((), ())),
                             preferred_element_type=jnp.float32)
    logits = logits + gb_ref[...]
    safe_temp = jnp.maximum(temp_ref[0, 0], 0.1)
    logits = logits / safe_temp
    m = jnp.max(logits, axis=0, keepdims=True)
    ex = jnp.exp(logits - m)
    probs = ex / jnp.sum(ex, axis=0, keepdims=True)
    probs_ref[...] = probs

    bits = lax.bitcast_convert_type(probs, jnp.int32)

    def it(_, lohi):
        lo, hi = lohi
        mid = lo + (hi - lo) // 2
        cnt = jnp.sum((bits >= mid).astype(jnp.int32), axis=1, keepdims=True)
        ge = cnt >= cap
        return jnp.where(ge, mid, lo), jnp.where(ge, hi, mid)

    lo0 = jnp.zeros((NUM_EXPERTS, 1), jnp.int32)
    hi0 = jnp.full((NUM_EXPERTS, 1), ONE_BITS, jnp.int32)
    lo, _ = lax.fori_loop(0, 31, it, (lo0, hi0))
    thr_ref[...] = jnp.broadcast_to(lo, (NUM_EXPERTS, 128))


def _router(x2, gate_w, gate_b, temperature, cap):
    T = x2.shape[0]
    return pl.pallas_call(
        functools.partial(_router_body, cap),
        out_shape=[jax.ShapeDtypeStruct((NUM_EXPERTS, T), jnp.float32),
                   jax.ShapeDtypeStruct((NUM_EXPERTS, 128), jnp.int32)],
    )(x2, gate_w, gate_b.reshape(NUM_EXPERTS, 1), temperature.reshape(1, 1))


_GCHUNK = 64


def _topk_gather_sc(probs_t, thr, x2, cap):
    T, D = x2.shape
    E = NUM_EXPERTS
    epc = E // 2
    mesh = plsc.VectorSubcoreMesh(core_axis_name="c", subcore_axis_name="s")

    cp = pltpu.CompilerParams()
    if "needs_layout_passes" in pltpu.CompilerParams.__dataclass_fields__:
        cp = dataclasses.replace(cp, needs_layout_passes=False)

    @functools.partial(
        pl.kernel,
        compiler_params=cp,
        out_type=[jax.ShapeDtypeStruct((E * cap, D), jnp.float32),
                  jax.ShapeDtypeStruct((E, cap), jnp.int32),
                  jax.ShapeDtypeStruct((E, cap), jnp.float32)],
        mesh=mesh,
        scratch_types=[
            pltpu.VMEM((T,), jnp.float32),
            pltpu.VMEM((cap + 32,), jnp.int32),
            pltpu.VMEM((cap + 32,), jnp.float32),
            pltpu.VMEM((cap + 32,), jnp.int32),
            pltpu.VMEM((16,), jnp.int32),
            pltpu.VMEM((2 * _GCHUNK,), jnp.int32),
            pltpu.VMEM((_GCHUNK // 2, D), jnp.float32),
            pltpu.VMEM((_GCHUNK // 2, D), jnp.float32),
            pltpu.SemaphoreType.DMA,
            pltpu.SemaphoreType.DMA,
        ],
    )
    def k(probs_hbm, th_hbm, x_hbm, xs_hbm, idx_hbm, sc_hbm,
          pf_v, si_v, sv_v, se_v, th_v, ic_v, rows_a, rows_b, sem, wsem):
        c = lax.axis_index("c")
        s = lax.axis_index("s")

        @pl.when(s < epc)
        def _():
            e = c * epc + s
            pltpu.sync_copy(probs_hbm.at[e], pf_v)
            pltpu.sync_copy(th_hbm.at[e, pl.ds(0, 16)], th_v)
            zvec = jnp.zeros((16,), jnp.int32)
            tvec = plsc.load_gather(th_v, [zvec])
            tfvec = plsc.bitcast(tvec, jnp.float32)
            lanes = jnp.arange(16, dtype=jnp.int32)

            def scan_chunk(j4, carry):
                off, eoff = carry
                for u in range(4):
                    j = j4 * 4 + u
                    pf = pf_v[pl.ds(j * 16, 16)]
                    pb = plsc.bitcast(pf, jnp.int32)
                    iv = lanes + j * 16
                    mg = pb > tvec
                    cg = jnp.max(plsc.all_reduce_population_count(mg))
                    plsc.store_compressed(si_v.at[pl.ds(off, 16)], iv, mask=mg)
                    plsc.store_compressed(sv_v.at[pl.ds(off, 16)], pf, mask=mg)
                    off = off + cg
                    me = pb == tvec
                    ce = jnp.max(plsc.all_reduce_population_count(me))

                    @pl.when(eoff < cap)
                    def _():
                        plsc.store_compressed(se_v.at[pl.ds(eoff, 16)], iv,
                                              mask=me)

                    eoff = eoff + jnp.where(eoff < cap, ce, 0)
                return off, eoff

            ngt, _ = lax.fori_loop(0, T // 64, scan_chunk, (0, 0))

            def fill(carry):
                j, off = carry
                si_v[pl.ds(off, 16)] = se_v[pl.ds(j * 16, 16)]
                sv_v[pl.ds(off, 16)] = tfvec
                return j + 1, off + 16

            lax.while_loop(lambda carry: carry[1] < cap, fill, (0, ngt))

            pltpu.sync_copy(si_v.at[pl.ds(0, cap)], idx_hbm.at[e])
            pltpu.sync_copy(sv_v.at[pl.ds(0, cap)], sc_hbm.at[e])

        plsc.subcore_barrier()
        eg = c * epc + s // 4
        part = s % 4
        nslice = 2 * _GCHUNK
        pltpu.sync_copy(idx_hbm.at[eg, pl.ds(part * nslice, nslice)], ic_v)

        base = eg * cap + part * nslice
        gc = _GCHUNK // 2
        bufs = (rows_a, rows_b)
        wbs = []
        for g in range(4):
            buf = bufs[g % 2]
            if g >= 2:
                wbs[g - 2].wait()
            pltpu.async_copy(x_hbm.at[ic_v.at[pl.ds(g * gc, gc)]],
                             buf, sem).wait()
            wbs.append(pltpu.async_copy(
                buf, xs_hbm.at[pl.ds(base + g * gc, gc)], wsem))
        wbs[2].wait()
        wbs[3].wait()

    return k(probs_t, thr, x2)


def _expert_body(xs_ref, ekw_ref, ekb_ref, evw_ref, evb_ref, sc_ref, out_ref):
    xs_bf = xs_ref[...].astype(jnp.bfloat16)
    kw_bf = ekw_ref[0].astype(jnp.bfloat16)
    h = jnp.dot(xs_bf, kw_bf, preferred_element_type=jnp.float32)
    h = jax.nn.gelu(h + ekb_ref[0])
    vw_bf = evw_ref[0].astype(jnp.bfloat16)
    o = jnp.dot(h.astype(jnp.bfloat16), vw_bf, preferred_element_type=jnp.float32)
    o = o + evb_ref[0]
    out_ref[...] = (o * sc_ref[0, 0][:, None]).astype(jnp.bfloat16)


def _routed_ffn(xs, ek_w, ek_b, ev_w, ev_b, scores, cap):
    E = NUM_EXPERTS
    return pl.pallas_call(
        _expert_body,
        grid=(E,),
        in_specs=[
            pl.BlockSpec((cap, D_MODEL), lambda e: (e, 0)),
            pl.BlockSpec((1, D_MODEL, HIDDEN), lambda e: (e, 0, 0)),
            pl.BlockSpec((1, 1, HIDDEN), lambda e: (e, 0, 0)),
            pl.BlockSpec((1, HIDDEN, D_MODEL), lambda e: (e, 0, 0)),
            pl.BlockSpec((1, 1, D_MODEL), lambda e: (e, 0, 0)),
            pl.BlockSpec((1, 1, cap), lambda e: (e, 0, 0)),
        ],
        out_specs=pl.BlockSpec((cap, D_MODEL), lambda e: (e, 0)),
        out_shape=jax.ShapeDtypeStruct((E * cap, D_MODEL), jnp.bfloat16),
    )(xs, ek_w, ek_b.reshape(E, 1, HIDDEN), ev_w,
      ev_b.reshape(E, 1, D_MODEL), scores.reshape(E, 1, cap))


def _shared_body(x_ref, skw_ref, skb_ref, svw_ref, svb_ref, out_ref):
    x_bf = x_ref[...].astype(jnp.bfloat16)
    kw_bf = skw_ref[...].astype(jnp.bfloat16)
    h = jnp.dot(x_bf, kw_bf, preferred_element_type=jnp.float32)
    h = jax.nn.gelu(h + skb_ref[...])
    vw_bf = svw_ref[...].astype(jnp.bfloat16)
    o = jnp.dot(h.astype(jnp.bfloat16), vw_bf, preferred_element_type=jnp.float32)
    out_ref[...] = o + svb_ref[...]


def _shared_ffn(x2, sk_w, sk_b, sv_w, sv_b, blk):
    T = x2.shape[0]
    return pl.pallas_call(
        _shared_body,
        grid=(T // blk,),
        in_specs=[
            pl.BlockSpec((blk, D_MODEL), lambda i: (i, 0)),
            pl.BlockSpec((D_MODEL, HIDDEN), lambda i: (0, 0)),
            pl.BlockSpec((1, HIDDEN), lambda i: (0, 0)),
            pl.BlockSpec((HIDDEN, D_MODEL), lambda i: (0, 0)),
            pl.BlockSpec((1, D_MODEL), lambda i: (0, 0)),
        ],
        out_specs=pl.BlockSpec((blk, D_MODEL), lambda i: (i, 0)),
        out_shape=jax.ShapeDtypeStruct((T, D_MODEL), jnp.float32),
    )(x2, sk_w, sk_b.reshape(1, HIDDEN), sv_w, sv_b.reshape(1, D_MODEL))


def kernel(x, gate_w, gate_b, temperature, ek_w, ek_b, ev_w, ev_b,
           sk_w, sk_b, sv_w, sv_b):
    B, S, D = x.shape
    T = B * S
    cap = max(8, T // NUM_EXPERTS)
    x2 = x.reshape(T, D)

    probs_t, thr = _router(x2, gate_w, gate_b, temperature, cap)

    xs, token_idx, scores = _topk_gather_sc(probs_t, thr, x2, cap)

    slots = _routed_ffn(xs, ek_w, ek_b, ev_w, ev_b, scores, cap)
    shared = _shared_ffn(x2, sk_w, sk_b, sv_w, sv_b, blk=512)

    out = shared.at[token_idx.reshape(-1)].add(slots, mode='promise_in_bounds')
    return out.reshape(B, S, D)
